# baseline (speedup 1.0000x reference)
import jax
import jax.numpy as jnp
from jax.experimental import pallas as pl
from jax.experimental.pallas import tpu as pltpu

_N_CHUNKS = 8


def _copy_body(emb_ref, out_ref, sem):
    rows = emb_ref.shape[0]
    chunk = rows // _N_CHUNKS
    for i in range(_N_CHUNKS):
        pltpu.make_async_copy(
            emb_ref.at[pl.ds(i * chunk, chunk)],
            out_ref.at[pl.ds(i * chunk, chunk)],
            sem,
        ).start()
    for i in range(_N_CHUNKS):
        pltpu.make_async_copy(
            emb_ref.at[pl.ds(i * chunk, chunk)],
            out_ref.at[pl.ds(i * chunk, chunk)],
            sem,
        ).wait()


def kernel(x, emb):
    seq_len = x.shape[1]
    d = emb.shape[1]
    return pl.pallas_call(
        _copy_body,
        in_specs=[pl.BlockSpec(memory_space=pl.ANY)],
        out_specs=pl.BlockSpec(memory_space=pl.ANY),
        out_shape=jax.ShapeDtypeStruct((seq_len, d), emb.dtype),
        scratch_shapes=[pltpu.SemaphoreType.DMA],
    )(emb)


# speedup vs baseline: 44.8817x; 44.8817x over previous
import jax
import jax.numpy as jnp
from jax.experimental import pallas as pl
from jax.experimental.pallas import tpu as pltpu

_BLOCK_ROWS = 1024


def _copy_block(emb_ref, out_ref):
    out_ref[...] = emb_ref[...]


def kernel(x, emb):
    seq_len = x.shape[1]
    d = emb.shape[1]
    grid = (seq_len // _BLOCK_ROWS,)
    return pl.pallas_call(
        _copy_block,
        grid=grid,
        in_specs=[pl.BlockSpec((_BLOCK_ROWS, d), lambda i: (i, 0))],
        out_specs=pl.BlockSpec((_BLOCK_ROWS, d), lambda i: (i, 0)),
        out_shape=jax.ShapeDtypeStruct((seq_len, d), emb.dtype),
    )(emb)


# speedup vs baseline: 48.5200x; 1.0811x over previous
import jax
import jax.numpy as jnp
from jax.experimental import pallas as pl
from jax.experimental.pallas import tpu as pltpu

_BLOCK_ROWS = 2048


def _copy_block(emb_ref, out_ref):
    out_ref[...] = emb_ref[...]


def kernel(x, emb):
    seq_len = x.shape[1]
    d = emb.shape[1]
    grid = (seq_len // _BLOCK_ROWS,)
    return pl.pallas_call(
        _copy_block,
        grid=grid,
        in_specs=[pl.BlockSpec((_BLOCK_ROWS, d), lambda i: (i, 0))],
        out_specs=pl.BlockSpec((_BLOCK_ROWS, d), lambda i: (i, 0)),
        out_shape=jax.ShapeDtypeStruct((seq_len, d), emb.dtype),
    )(emb)
